# baseline (speedup 1.0000x reference)
import functools

import jax
import jax.numpy as jnp
from jax.experimental import pallas as pl
from jax.experimental.pallas import tpu as pltpu

C = 64
H = W = 28
HW = H * W
PAD = 32
SEG = HW + 2 * PAD


def _conv_body(apply_act, g_imgs, x_ref, w_ref, b_ref, s_ref, t_ref,
               out_ref, stat_ref, buf_ref):
    step = pl.program_id(0)

    @pl.when(step == 0)
    def _init():
        stat_ref[...] = jnp.zeros_like(stat_ref)
        buf_ref[...] = jnp.zeros_like(buf_ref)

    x = x_ref[...]
    s = s_ref[...][None]
    t = t_ref[...][None]
    xn = x * s + t
    if apply_act:
        xn = jnp.maximum(xn, 0.0)

    col = jax.lax.broadcasted_iota(jnp.int32, (1, HW), 1) % W
    m_left = jnp.where(col == W - 1, 0.0, 1.0)
    m_right = jnp.where(col == 0, 0.0, 1.0)

    for g in range(g_imgs):
        xg = xn[g]
        base = PAD + g * SEG
        buf_ref[0, :, base:base + HW] = xg * m_left
        buf_ref[1, :, base:base + HW] = xg
        buf_ref[2, :, base:base + HW] = xg * m_right

    wtot = g_imgs * SEG - 2 * PAD
    acc = None
    for ky in range(3):
        for kx in range(3):
            off = (ky - 1) * W + (kx - 1)
            xs = buf_ref[kx, :, PAD + off:PAD + off + wtot]
            part = jnp.dot(w_ref[ky * 3 + kx], xs,
                           preferred_element_type=jnp.float32)
            acc = part if acc is None else acc + part
    y = acc + b_ref[...]

    ssum = None
    ssq = None
    for g in range(g_imgs):
        yg = y[:, g * SEG:g * SEG + HW]
        if out_ref.shape[0] == g_imgs:
            out_ref[g] = yg
        else:
            out_ref[:, g, :] = yg
        ps = jnp.sum(yg, axis=1)
        pq = jnp.sum(yg * yg, axis=1)
        ssum = ps if ssum is None else ssum + ps
        ssq = pq if ssq is None else ssq + pq
    stat_ref[0:1, :] = stat_ref[0:1, :] + ssum[None]
    stat_ref[1:2, :] = stat_ref[1:2, :] + ssq[None]


def _conv_layer(x, wtaps, b, s, t, apply_act, out_cmajor, g_imgs):
    batch = x.shape[0]
    grid = batch // g_imgs
    out_shape = (C, batch, HW) if out_cmajor else (batch, C, HW)
    out_spec = (pl.BlockSpec((C, g_imgs, HW), lambda i: (0, i, 0))
                if out_cmajor else
                pl.BlockSpec((g_imgs, C, HW), lambda i: (i, 0, 0)))
    return pl.pallas_call(
        functools.partial(_conv_body, apply_act, g_imgs),
        grid=(grid,),
        in_specs=[
            pl.BlockSpec((g_imgs, C, HW), lambda i: (i, 0, 0)),
            pl.BlockSpec((9, C, C), lambda i: (0, 0, 0)),
            pl.BlockSpec((C, 1), lambda i: (0, 0)),
            pl.BlockSpec((C, 1), lambda i: (0, 0)),
            pl.BlockSpec((C, 1), lambda i: (0, 0)),
        ],
        out_specs=[out_spec, pl.BlockSpec((8, C), lambda i: (0, 0))],
        out_shape=[jax.ShapeDtypeStruct(out_shape, jnp.float32),
                   jax.ShapeDtypeStruct((8, C), jnp.float32)],
        scratch_shapes=[pltpu.VMEM((3, C, g_imgs * SEG), jnp.float32)],
    )(x, wtaps, b, s, t)


def _fc_body(g_imgs, x_ref, w_ref, b_ref, s_ref, t_ref, out_ref, stat_ref):
    step = pl.program_id(0)

    @pl.when(step == 0)
    def _init():
        stat_ref[...] = jnp.zeros_like(stat_ref)

    x = x_ref[...]
    s = s_ref[...][:, :, None]
    t = t_ref[...][:, :, None]
    xn = jnp.maximum(x * s + t, 0.0)
    w = w_ref[...]
    b = b_ref[...]
    ssum = None
    ssq = None
    for g in range(g_imgs):
        e = jnp.dot(w, xn[:, g, :], preferred_element_type=jnp.float32) + b
        out_ref[:, g, :] = e
        ps = jnp.sum(e, axis=1)
        pq = jnp.sum(e * e, axis=1)
        ssum = ps if ssum is None else ssum + ps
        ssq = pq if ssq is None else ssq + pq
    stat_ref[0:1, :] = stat_ref[0:1, :] + ssum[None]
    stat_ref[1:2, :] = stat_ref[1:2, :] + ssq[None]


def _fc_layer(x, w, b, s, t, g_imgs):
    batch = x.shape[1]
    grid = batch // g_imgs
    return pl.pallas_call(
        functools.partial(_fc_body, g_imgs),
        grid=(grid,),
        in_specs=[
            pl.BlockSpec((C, g_imgs, HW), lambda i: (0, i, 0)),
            pl.BlockSpec((C, C), lambda i: (0, 0)),
            pl.BlockSpec((C, 1), lambda i: (0, 0)),
            pl.BlockSpec((C, 1), lambda i: (0, 0)),
            pl.BlockSpec((C, 1), lambda i: (0, 0)),
        ],
        out_specs=[pl.BlockSpec((C, g_imgs, HW), lambda i: (0, i, 0)),
                   pl.BlockSpec((8, C), lambda i: (0, 0))],
        out_shape=[jax.ShapeDtypeStruct((C, batch, HW), jnp.float32),
                   jax.ShapeDtypeStruct((8, C), jnp.float32)],
    )(x, w, b, s, t)


def _final_body(g_imgs, x_ref, s_ref, t_ref, out_ref):
    x = x_ref[...]
    s = s_ref[...][:, :, None]
    t = t_ref[...][:, :, None]
    y = jnp.maximum(x * s + t, 0.0)
    nrm = jnp.sqrt(jnp.sum(y * y, axis=0, keepdims=True))
    e = y / (nrm + 1e-8)
    for g in range(g_imgs):
        out_ref[g] = e[:, g, :].T


def _final_layer(x, s, t, g_imgs):
    batch = x.shape[1]
    grid = batch // g_imgs
    return pl.pallas_call(
        functools.partial(_final_body, g_imgs),
        grid=(grid,),
        in_specs=[
            pl.BlockSpec((C, g_imgs, HW), lambda i: (0, i, 0)),
            pl.BlockSpec((C, 1), lambda i: (0, 0)),
            pl.BlockSpec((C, 1), lambda i: (0, 0)),
        ],
        out_specs=pl.BlockSpec((g_imgs, HW, C), lambda i: (i, 0, 0)),
        out_shape=jax.ShapeDtypeStruct((batch, HW, C), jnp.float32),
    )(x, s, t)


def _fold_bn(stat, gamma, beta, n):
    mu = stat[0] / n
    var = stat[1] / n - mu * mu
    scale = gamma * jax.lax.rsqrt(var + 1e-5)
    shift = beta - mu * scale
    return scale.reshape(C, 1), shift.reshape(C, 1)


def _taps(w):
    return jnp.transpose(w, (2, 3, 0, 1)).reshape(9, C, C)


def kernel(x, conv0a_w, conv0a_b, bn0a_g, bn0a_b, conv0b_w, conv0b_b, bn0b_g, bn0b_b,
           conv1a_w, conv1a_b, bn1a_g, bn1a_b, conv1b_w, conv1b_b, bn1b_g, bn1b_b,
           fc0_w, fc0_b, bnf0_g, bnf0_b, fc1_w, fc1_b, bnf1_g, bnf1_b):
    batch = x.shape[0]
    g_imgs = 8 if batch % 8 == 0 else 1
    n = batch * HW
    x3 = x.reshape(batch, C, HW)
    ones = jnp.ones((C, 1), jnp.float32)
    zeros = jnp.zeros((C, 1), jnp.float32)

    y, st = _conv_layer(x3, _taps(conv0a_w), conv0a_b.reshape(C, 1),
                        ones, zeros, False, False, g_imgs)
    s, t = _fold_bn(st, bn0a_g, bn0a_b, n)
    y, st = _conv_layer(y, _taps(conv0b_w), conv0b_b.reshape(C, 1),
                        s, t, True, False, g_imgs)
    s, t = _fold_bn(st, bn0b_g, bn0b_b, n)
    y, st = _conv_layer(y, _taps(conv1a_w), conv1a_b.reshape(C, 1),
                        s, t, True, False, g_imgs)
    s, t = _fold_bn(st, bn1a_g, bn1a_b, n)
    y, st = _conv_layer(y, _taps(conv1b_w), conv1b_b.reshape(C, 1),
                        s, t, True, True, g_imgs)
    s, t = _fold_bn(st, bn1b_g, bn1b_b, n)

    e, st = _fc_layer(y, fc0_w, fc0_b.reshape(C, 1), s, t, g_imgs)
    s, t = _fold_bn(st, bnf0_g, bnf0_b, n)
    e, st = _fc_layer(e, fc1_w, fc1_b.reshape(C, 1), s, t, g_imgs)
    s, t = _fold_bn(st, bnf1_g, bnf1_b, n)

    out = _final_layer(e, s, t, g_imgs)
    return out.reshape(batch * HW, C)
